# Initial kernel scaffold; baseline (speedup 1.0000x reference)
import jax
import jax.numpy as jnp
from jax.experimental import pallas as pl


def kernel(means, quaternions, weights, indices, ctrl_translations, ctrl_rotations, ctrl_positions):
    raise NotImplementedError("write your pallas kernel here")


# speedup vs baseline: 13.1020x; 13.1020x over previous
import functools

import jax
import jax.numpy as jnp
from jax import lax
from jax.experimental import pallas as pl
from jax.experimental.pallas import tpu as pltpu
from jax.experimental.pallas import tpu_sc as plsc

N = 100000
K_NB = 10
K_CTRL = 80
EPS = 1e-8
LANES = 16
GROUPS_PER_WORKER = 196
ROWS_PER_WORKER = GROUPS_PER_WORKER * LANES
NUM_WORKERS = 32
LAST_START = N - ROWS_PER_WORKER


def _rsqrt(x):
    x = jnp.maximum(x, jnp.float32(1e-30))
    i = plsc.bitcast(x, jnp.int32)
    i = jnp.int32(0x5F3759DF) - (i >> 1)
    y = plsc.bitcast(i, jnp.float32)
    for _ in range(3):
        y = y * (jnp.float32(1.5) - jnp.float32(0.5) * x * y * y)
    return y


def _sc_body(means_h, quats_h, w_h, idx_h, crot_h, cpos_h, ctr_h,
             outm_h, outq_h,
             idx_v, w_v, means_v, quats_v, outm_v, outq_v,
             pay_v, crot_v, cpos_v, ctr_v):
    wid = lax.axis_index("s") * 2 + lax.axis_index("c")
    s_row = jnp.minimum(wid * ROWS_PER_WORKER, LAST_START)

    pltpu.sync_copy(idx_h.at[pl.ds(s_row * K_NB, ROWS_PER_WORKER * K_NB)], idx_v)
    pltpu.sync_copy(w_h.at[pl.ds(s_row * K_NB, ROWS_PER_WORKER * K_NB)], w_v)
    pltpu.sync_copy(means_h.at[pl.ds(s_row * 3, ROWS_PER_WORKER * 3)], means_v)
    pltpu.sync_copy(quats_h.at[pl.ds(s_row * 4, ROWS_PER_WORKER * 4)], quats_v)
    pltpu.sync_copy(crot_h, crot_v)
    pltpu.sync_copy(cpos_h, cpos_v)
    pltpu.sync_copy(ctr_h, ctr_v)

    lane = lax.iota(jnp.int32, LANES)

    for cg in range(K_CTRL // LANES):
        c = cg * LANES + lane
        c4 = c * 4
        c3 = c * 3
        qw = plsc.load_gather(crot_v, [c4])
        qx = plsc.load_gather(crot_v, [c4 + 1])
        qy = plsc.load_gather(crot_v, [c4 + 2])
        qz = plsc.load_gather(crot_v, [c4 + 3])
        n2 = qw * qw + qx * qx + qy * qy + qz * qz
        inv = jnp.float32(1.0) / (n2 * _rsqrt(n2) + jnp.float32(EPS))
        nw, nx, ny, nz = qw * inv, qx * inv, qy * inv, qz * inv
        two = jnp.float32(2.0)
        one = jnp.float32(1.0)
        r00 = one - two * (ny * ny + nz * nz)
        r01 = two * (nx * ny - nw * nz)
        r02 = two * (nx * nz + nw * ny)
        r10 = two * (nx * ny + nw * nz)
        r11 = one - two * (nx * nx + nz * nz)
        r12 = two * (ny * nz - nw * nx)
        r20 = two * (nx * nz - nw * ny)
        r21 = two * (ny * nz + nw * nx)
        r22 = one - two * (nx * nx + ny * ny)
        px = plsc.load_gather(cpos_v, [c3])
        py = plsc.load_gather(cpos_v, [c3 + 1])
        pz = plsc.load_gather(cpos_v, [c3 + 2])
        tx = plsc.load_gather(ctr_v, [c3])
        ty = plsc.load_gather(ctr_v, [c3 + 1])
        tz = plsc.load_gather(ctr_v, [c3 + 2])
        b0 = px + tx - (r00 * px + r01 * py + r02 * pz)
        b1 = py + ty - (r10 * px + r11 * py + r12 * pz)
        b2 = pz + tz - (r20 * px + r21 * py + r22 * pz)
        p16 = c * 16
        payload = (r00, r01, r02, r10, r11, r12, r20, r21, r22,
                   b0, b1, b2, qw, qx, qy, qz)
        for j, val in enumerate(payload):
            plsc.store_scatter(pay_v, [p16 + j], val)

    def group(g, carry):
        r = g * LANES + lane
        riw = r * K_NB
        acc = [None] * 16
        for k in range(K_NB):
            ik = plsc.load_gather(idx_v, [riw + k])
            wk = plsc.load_gather(w_v, [riw + k])
            p = ik * 16
            for j in range(16):
                v = plsc.load_gather(pay_v, [p + j])
                t = wk * v
                acc[j] = t if k == 0 else acc[j] + t
        r3 = r * 3
        r4 = r * 4
        mx = plsc.load_gather(means_v, [r3])
        my = plsc.load_gather(means_v, [r3 + 1])
        mz = plsc.load_gather(means_v, [r3 + 2])
        ox = acc[0] * mx + acc[1] * my + acc[2] * mz + acc[9]
        oy = acc[3] * mx + acc[4] * my + acc[5] * mz + acc[10]
        oz = acc[6] * mx + acc[7] * my + acc[8] * mz + acc[11]
        plsc.store_scatter(outm_v, [r3], ox)
        plsc.store_scatter(outm_v, [r3 + 1], oy)
        plsc.store_scatter(outm_v, [r3 + 2], oz)
        bw, bx, by, bz = acc[12], acc[13], acc[14], acc[15]
        n2 = bw * bw + bx * bx + by * by + bz * bz
        inv = jnp.float32(1.0) / (n2 * _rsqrt(n2) + jnp.float32(EPS))
        bw, bx, by, bz = bw * inv, bx * inv, by * inv, bz * inv
        qw = plsc.load_gather(quats_v, [r4])
        qx = plsc.load_gather(quats_v, [r4 + 1])
        qy = plsc.load_gather(quats_v, [r4 + 2])
        qz = plsc.load_gather(quats_v, [r4 + 3])
        ow = bw * qw - bx * qx - by * qy - bz * qz
        oxq = bw * qx + bx * qw + by * qz - bz * qy
        oyq = bw * qy - bx * qz + by * qw + bz * qx
        ozq = bw * qz + bx * qy - by * qx + bz * qw
        plsc.store_scatter(outq_v, [r4], ow)
        plsc.store_scatter(outq_v, [r4 + 1], oxq)
        plsc.store_scatter(outq_v, [r4 + 2], oyq)
        plsc.store_scatter(outq_v, [r4 + 3], ozq)
        return carry

    lax.fori_loop(0, GROUPS_PER_WORKER, group, 0)

    pltpu.sync_copy(outm_v, outm_h.at[pl.ds(s_row * 3, ROWS_PER_WORKER * 3)])
    pltpu.sync_copy(outq_v, outq_h.at[pl.ds(s_row * 4, ROWS_PER_WORKER * 4)])


@functools.partial(
    pl.kernel,
    out_type=[
        jax.ShapeDtypeStruct((N * 3,), jnp.float32),
        jax.ShapeDtypeStruct((N * 4,), jnp.float32),
    ],
    mesh=plsc.VectorSubcoreMesh(core_axis_name="c", subcore_axis_name="s"),
    compiler_params=pltpu.CompilerParams(needs_layout_passes=False),
    scratch_types=[
        pltpu.VMEM((ROWS_PER_WORKER * K_NB,), jnp.int32),
        pltpu.VMEM((ROWS_PER_WORKER * K_NB,), jnp.float32),
        pltpu.VMEM((ROWS_PER_WORKER * 3,), jnp.float32),
        pltpu.VMEM((ROWS_PER_WORKER * 4,), jnp.float32),
        pltpu.VMEM((ROWS_PER_WORKER * 3,), jnp.float32),
        pltpu.VMEM((ROWS_PER_WORKER * 4,), jnp.float32),
        pltpu.VMEM((K_CTRL * 16,), jnp.float32),
        pltpu.VMEM((K_CTRL * 4,), jnp.float32),
        pltpu.VMEM((K_CTRL * 3,), jnp.float32),
        pltpu.VMEM((K_CTRL * 3,), jnp.float32),
    ],
)
def _lbs_sc(means_h, quats_h, w_h, idx_h, crot_h, cpos_h, ctr_h,
            outm_h, outq_h, *scratch):
    _sc_body(means_h, quats_h, w_h, idx_h, crot_h, cpos_h, ctr_h,
             outm_h, outq_h, *scratch)


def kernel(means, quaternions, weights, indices,
           ctrl_translations, ctrl_rotations, ctrl_positions):
    outm, outq = _lbs_sc(
        means.reshape(-1).astype(jnp.float32),
        quaternions.reshape(-1).astype(jnp.float32),
        weights.reshape(-1).astype(jnp.float32),
        indices.reshape(-1).astype(jnp.int32),
        ctrl_rotations.reshape(-1).astype(jnp.float32),
        ctrl_positions.reshape(-1).astype(jnp.float32),
        ctrl_translations.reshape(-1).astype(jnp.float32),
    )
    return outm.reshape(N, 3), outq.reshape(N, 4)
